# baseline (speedup 1.0000x reference)
import functools

import jax
import jax.numpy as jnp
from jax import lax
from jax.experimental import pallas as pl
from jax.experimental.pallas import tpu as pltpu
from jax.experimental.pallas import tpu_sc as plsc

VOCAB = 100000
D = 128

NC = 2
NS = 16
NW = NC * NS

B = 4096 * 200
B_PER_W = B // NW
CHUNK = 128
N_CHUNKS = B_PER_W // CHUNK
GPW = 2
WROWS = GPW * CHUNK
N_W = B_PER_W // WROWS
NBUF = 3


def _make_kernel():
  mesh = plsc.VectorSubcoreMesh(core_axis_name="c", subcore_axis_name="s")

  @functools.partial(
      pl.kernel,
      out_type=jax.ShapeDtypeStruct((B, D), jnp.float32),
      mesh=mesh,
      scratch_types=[
          pltpu.VMEM((N_CHUNKS, CHUNK), jnp.int32),
          [pltpu.VMEM((WROWS, D), jnp.float32)] * NBUF,
          [pltpu.SemaphoreType.DMA] * NBUF,
          [pltpu.SemaphoreType.DMA] * NBUF,
      ],
  )
  def k(x_hbm, table_hbm, out_hbm, idx_v, rows, gs, ws):
    wid = lax.axis_index("s") * NC + lax.axis_index("c")
    base = wid * B_PER_W
    pltpu.sync_copy(x_hbm.at[pl.ds(wid * N_CHUNKS, N_CHUNKS)], idx_v)

    def gather(c, b):
      for j in range(GPW):
        pltpu.async_copy(table_hbm.at[idx_v.at[GPW * c + j]],
                         rows[b].at[pl.ds(j * CHUNK, CHUNK)], gs[b])

    def wait_gather(b):
      for j in range(GPW):
        pltpu.make_async_copy(table_hbm.at[idx_v.at[0]],
                              rows[b].at[pl.ds(0, CHUNK)], gs[b]).wait()

    def write(c, b):
      pltpu.async_copy(rows[b], out_hbm.at[pl.ds(base + c * WROWS, WROWS)],
                       ws[b])

    def wait_write(b):
      pltpu.make_async_copy(rows[b], out_hbm.at[pl.ds(base, WROWS)],
                            ws[b]).wait()

    gather(0, 0)
    for c in range(NBUF - 1):
      wait_gather(c)
      gather(c + 1, c + 1)
      write(c, c)

    steady0 = NBUF - 1
    n_steady = N_W - NBUF
    n_loop = (n_steady // NBUF) * NBUF

    def step(c, b):
      wait_gather(b)
      wait_write((b + 1) % NBUF)
      gather(c + 1, (b + 1) % NBUF)
      write(c, b)

    def body(t, carry):
      for u in range(NBUF):
        step(steady0 + NBUF * t + u, (steady0 + u) % NBUF)
      return carry

    lax.fori_loop(0, n_loop // NBUF, body, 0)
    for i in range(n_steady - n_loop):
      c = steady0 + n_loop + i
      step(c, c % NBUF)

    c = N_W - 1
    wait_gather(c % NBUF)
    wait_write((c + 1) % NBUF)
    write(c, c % NBUF)
    wait_write((c + NBUF - 1) % NBUF)
    wait_write(c % NBUF)

  return k


_kernel = _make_kernel()


def kernel(x, table):
  out = _kernel(x.reshape(B // CHUNK, CHUNK).astype(jnp.int32), table)
  return out.reshape(x.shape[0], x.shape[1], D)
